# baseline (speedup 1.0000x reference)
import functools

import jax
import jax.numpy as jnp
from jax import lax
from jax.experimental import pallas as pl
from jax.experimental.pallas import tpu as pltpu
from jax.experimental.pallas import tpu_sc as plsc

D_MODEL = 64
SCALE = 8.0

NC = 2
NS = 16
LANES = 16
NW = NC * NS

VOCAB_MAIN = 999936
N_GROUPS = VOCAB_MAIN // 128

SUP = 1024
GCH = 128
NG = SUP // GCH


@functools.lru_cache(maxsize=None)
def _make_detile(vocab):
    n_tail = vocab - VOCAB_MAIN
    mesh = plsc.VectorSubcoreMesh(
        core_axis_name="c", subcore_axis_name="s",
        num_cores=NC, num_subcores=NS)
    base_groups = N_GROUPS // NW
    extra = N_GROUPS - base_groups * NW

    @functools.partial(
        pl.kernel,
        mesh=mesh,
        out_type=jax.ShapeDtypeStruct((vocab * D_MODEL,), jnp.float32),
        scratch_types=[
            pltpu.VMEM((8, 8, 128), jnp.float32),
            pltpu.VMEM((128 * D_MODEL,), jnp.float32),
            pltpu.VMEM((n_tail * D_MODEL,), jnp.float32),
            pltpu.SemaphoreType.DMA,
        ],
        compiler_params=pltpu.CompilerParams(
            use_tc_tiling_on_sc=True, needs_layout_passes=False),
    )
    def detile(tt_hbm, tail_hbm, out_hbm, tiles_v, rows_v, tail_v, sem):
        wid = lax.axis_index("s") * NC + lax.axis_index("c")
        base_g = wid * base_groups + jnp.minimum(wid, extra)
        n_g = base_groups + jnp.where(wid < extra, 1, 0)

        def group_body(g, carry):
            gid = base_g + g
            copies = [
                pltpu.async_copy(
                    tt_hbm.at[pl.ds(jh * 8, 8), pl.ds(gid * 128, 128)],
                    tiles_v.at[jh], sem)
                for jh in range(8)
            ]
            for cp in copies:
                cp.wait()

            lane = lax.iota(jnp.int32, LANES)

            def ch_body(j, c2):
                jh = j // 8
                jl = j % 8
                for il in range(8):
                    val = tiles_v[jh, jl, pl.ds(il * LANES, LANES)]
                    plsc.store_scatter(
                        rows_v,
                        [lane * D_MODEL + (il * LANES * D_MODEL + j)],
                        val * SCALE)
                return c2

            lax.fori_loop(0, D_MODEL, ch_body, 0)
            pltpu.sync_copy(
                rows_v, out_hbm.at[pl.ds(gid * 128 * D_MODEL, 128 * D_MODEL)])
            return carry

        lax.fori_loop(0, n_g, group_body, 0)

        @pl.when(wid == NW - 1)
        def _():
            pltpu.sync_copy(tail_hbm, tail_v)

            def tail_scale(i, c2):
                tail_v[pl.ds(i * LANES, LANES)] = (
                    tail_v[pl.ds(i * LANES, LANES)] * SCALE)
                return c2

            lax.fori_loop(0, n_tail * D_MODEL // LANES, tail_scale, 0)
            pltpu.sync_copy(
                tail_v,
                out_hbm.at[pl.ds(VOCAB_MAIN * D_MODEL, n_tail * D_MODEL)])

    return detile


@functools.lru_cache(maxsize=None)
def _make_lookup(n, vocab):
    b_per_w = n // NW
    n_sup = b_per_w // SUP
    mesh = plsc.VectorSubcoreMesh(
        core_axis_name="c", subcore_axis_name="s",
        num_cores=NC, num_subcores=NS)

    @functools.partial(
        pl.kernel,
        mesh=mesh,
        out_type=jax.ShapeDtypeStruct((n, D_MODEL), jnp.float32),
        scratch_types=[
            pltpu.VMEM((SUP,), jnp.int32),
            pltpu.VMEM((SUP, D_MODEL), jnp.float32),
            pltpu.SemaphoreType.DMA,
        ],
        compiler_params=pltpu.CompilerParams(use_tc_tiling_on_sc=False),
    )
    def lookup(table_hbm, idx_hbm, out_hbm, idx_v, rows_v, sem):
        wid = lax.axis_index("s") * NC + lax.axis_index("c")
        base = wid * b_per_w

        def sup_body(g, carry):
            off = base + g * SUP
            pltpu.sync_copy(idx_hbm.at[pl.ds(off, SUP)], idx_v)
            copies = [
                pltpu.async_copy(
                    table_hbm.at[idx_v.at[pl.ds(j * GCH, GCH)]],
                    rows_v.at[pl.ds(j * GCH, GCH)],
                    sem,
                )
                for j in range(NG)
            ]
            for cp in copies:
                cp.wait()
            pltpu.sync_copy(rows_v, out_hbm.at[pl.ds(off, SUP)])
            return carry

        lax.fori_loop(0, n_sup, sup_body, 0)

    return lookup


def kernel(x, table):
    b, l = x.shape
    vocab = table.shape[0]
    idx = x.reshape(b * l).astype(jnp.int32)
    tail = table[VOCAB_MAIN:].reshape((vocab - VOCAB_MAIN) * D_MODEL)
    flat = _make_detile(vocab)(table.T, tail)
    table_lin = flat.reshape(vocab, D_MODEL)
    out = _make_lookup(b * l, vocab)(table_lin, idx)
    return out.reshape(b, l, D_MODEL)
